# Initial kernel scaffold; baseline (speedup 1.0000x reference)
import jax
import jax.numpy as jnp
from jax.experimental import pallas as pl


def kernel(token_ids, track_ids, token_table, track_table, pe):
    raise NotImplementedError("write your pallas kernel here")


# speedup vs baseline: 3.2380x; 3.2380x over previous
import functools

import jax
import jax.numpy as jnp
from jax import lax
from jax.experimental import pallas as pl
from jax.experimental.pallas import tpu as pltpu
from jax.experimental.pallas import tpu_sc as plsc

_info = plsc.get_sparse_core_info()
_NC, _NS, _L = _info.num_cores, _info.num_subcores, _info.num_lanes
_NW = _NC * _NS
_SW = 16
_BW = _NW // _SW
_G = 64


def _combine_body(tok_ref, trk_ref, out_ref):
    t = tok_ref[...]
    out_ref[0] = t + trk_ref[0:1, :]
    out_ref[1] = t + trk_ref[1:2, :]


def _make_sc_call(B, S, V, D):
    SCH = S // _SW
    BCH = B // _BW
    H = SCH // _G
    mesh = plsc.VectorSubcoreMesh(core_axis_name="c", subcore_axis_name="s")

    def _sc_body(tok_hbm, trk_hbm, ctab_hbm, pe_hbm, out_hbm,
                 tok_v, trk_v, idx_v, pe_v, buf, g_sem):
        wid = lax.axis_index("s") * _NC + lax.axis_index("c")
        sidx = wid % _SW
        s0 = pl.multiple_of(sidx * SCH, SCH)
        b0 = pl.multiple_of((wid // _SW) * BCH, BCH)

        pltpu.sync_copy(tok_hbm.at[pl.ds(b0, BCH), pl.ds(s0, SCH)], tok_v)
        pltpu.sync_copy(trk_hbm.at[pl.ds(b0, BCH), pl.ds(s0, SCH)], trk_v)

        def idx_body(i, carry):
            for c in range(SCH // _L):
                sl = pl.ds(c * _L, _L)
                idx_v[i, sl] = trk_v[i, sl] * V + tok_v[i, sl]
            return carry

        lax.fori_loop(0, BCH, idx_body, 0)

        for h in range(H):
            pltpu.sync_copy(pe_hbm.at[pl.ds(s0 + h * _G, _G)], pe_v)

            def b_body(b, carry):
                pltpu.async_copy(
                    ctab_hbm.at[idx_v.at[b, pl.ds(h * _G, _G)]], buf, g_sem
                ).wait()

                def add_body(i, c2):
                    for c in range(D // _L):
                        sl = pl.ds(c * _L, _L)
                        plsc.addupdate(buf.at[i, sl], pe_v[i, sl])
                    return c2

                lax.fori_loop(0, _G, add_body, 0)
                r0 = (b0 + b) * S + s0 + h * _G
                pltpu.sync_copy(buf, out_hbm.at[pl.ds(r0, _G)])
                return carry

            lax.fori_loop(0, BCH, b_body, 0)

    return pl.kernel(
        _sc_body,
        out_type=jax.ShapeDtypeStruct((B * S, D), jnp.float32),
        mesh=mesh,
        scratch_types=[
            pltpu.VMEM((BCH, SCH), jnp.int32),
            pltpu.VMEM((BCH, SCH), jnp.int32),
            pltpu.VMEM((BCH, SCH), jnp.int32),
            pltpu.VMEM((_G, D), jnp.float32),
            pltpu.VMEM((_G, D), jnp.float32),
            pltpu.SemaphoreType.DMA,
        ],
    )


def kernel(token_ids, track_ids, token_table, track_table, pe):
    B, S = token_ids.shape
    V, D = token_table.shape
    T = track_table.shape[0]

    tok = token_ids.astype(jnp.int32)
    trk = track_ids.astype(jnp.int32)

    ctab = pl.pallas_call(
        _combine_body,
        out_shape=jax.ShapeDtypeStruct((T, V, D), jnp.float32),
    )(token_table, track_table)
    ctab = ctab.reshape(T * V, D)

    pe2d = pe.reshape(pe.shape[1], D)[:S]

    out = _make_sc_call(B, S, V, D)(tok, trk, ctab, pe2d)
    return out.reshape(B, S, D)


# speedup vs baseline: 5.0687x; 1.5654x over previous
import functools

import jax
import jax.numpy as jnp
from jax import lax
from jax.experimental import pallas as pl
from jax.experimental.pallas import tpu as pltpu
from jax.experimental.pallas import tpu_sc as plsc

_info = plsc.get_sparse_core_info()
_NC, _NS, _L = _info.num_cores, _info.num_subcores, _info.num_lanes
_NW = _NC * _NS
_SW = 16
_BW = _NW // _SW
_G = 64


def _combine_body(tok_ref, trk_ref, out_ref):
    t = tok_ref[...]
    out_ref[0] = t + trk_ref[0:1, :]
    out_ref[1] = t + trk_ref[1:2, :]


def _make_sc_call(B, S, V, D):
    SCH = S // _SW
    BCH = B // _BW
    H = SCH // _G
    mesh = plsc.VectorSubcoreMesh(core_axis_name="c", subcore_axis_name="s")

    def _sc_body(tok_hbm, trk_hbm, ctab_hbm, pe_hbm, out_hbm,
                 idx_v, trk_v, pe_v, buf0, buf1, g_sem, w_sem):
        wid = lax.axis_index("s") * _NC + lax.axis_index("c")
        s0 = pl.multiple_of((wid % _SW) * SCH, SCH)
        b0 = pl.multiple_of((wid // _SW) * BCH, BCH)

        pltpu.sync_copy(tok_hbm.at[pl.ds(b0, BCH), pl.ds(s0, SCH)], idx_v)
        pltpu.sync_copy(trk_hbm.at[pl.ds(b0, BCH), pl.ds(s0, SCH)], trk_v)

        def idx_body(i, carry):
            for c in range(SCH // _L):
                sl = pl.ds(c * _L, _L)
                idx_v[i, sl] = trk_v[i, sl] * V + idx_v[i, sl]
            return carry

        lax.fori_loop(0, BCH, idx_body, 0)

        def _wait_gather(buf):
            pltpu.make_async_copy(ctab_hbm.at[pl.ds(0, _G)], buf, g_sem).wait()

        def _wait_write(buf):
            pltpu.make_async_copy(buf, out_hbm.at[pl.ds(0, _G)], w_sem).wait()

        def _add_and_write(b, buf, h):
            def add_body(i, c2):
                for c in range(D // _L):
                    sl = pl.ds(c * _L, _L)
                    plsc.addupdate(buf.at[i, sl], pe_v[i, sl])
                return c2

            lax.fori_loop(0, _G, add_body, 0)
            r0 = (b0 + b) * S + s0 + h * _G
            pltpu.async_copy(buf, out_hbm.at[pl.ds(r0, _G)], w_sem)

        def _start_gather(b, buf, h):
            pltpu.async_copy(
                ctab_hbm.at[idx_v.at[b, pl.ds(h * _G, _G)]], buf, g_sem
            )

        for h in range(H):
            _start_gather(0, buf0, h)
            pltpu.sync_copy(pe_hbm.at[pl.ds(s0 + h * _G, _G)], pe_v)

            def pair_body(k, carry):
                b = 2 * k
                _wait_gather(buf0)

                @pl.when(k >= 1)
                def _():
                    _wait_write(buf1)

                _start_gather(b + 1, buf1, h)
                _add_and_write(b, buf0, h)

                _wait_gather(buf1)

                @pl.when(k < BCH // 2 - 1)
                def _():
                    _wait_write(buf0)
                    _start_gather(b + 2, buf0, h)

                _add_and_write(b + 1, buf1, h)
                return carry

            lax.fori_loop(0, BCH // 2, pair_body, 0)
            _wait_write(buf0)
            _wait_write(buf1)

    return pl.kernel(
        _sc_body,
        out_type=jax.ShapeDtypeStruct((B * S, D), jnp.float32),
        mesh=mesh,
        scratch_types=[
            pltpu.VMEM((BCH, SCH), jnp.int32),
            pltpu.VMEM((BCH, SCH), jnp.int32),
            pltpu.VMEM((_G, D), jnp.float32),
            pltpu.VMEM((_G, D), jnp.float32),
            pltpu.VMEM((_G, D), jnp.float32),
            pltpu.SemaphoreType.DMA,
            pltpu.SemaphoreType.DMA,
        ],
    )


def kernel(token_ids, track_ids, token_table, track_table, pe):
    B, S = token_ids.shape
    V, D = token_table.shape
    T = track_table.shape[0]

    tok = token_ids.astype(jnp.int32)
    trk = track_ids.astype(jnp.int32)

    ctab = pl.pallas_call(
        _combine_body,
        out_shape=jax.ShapeDtypeStruct((T, V, D), jnp.float32),
    )(token_table, track_table)
    ctab = ctab.reshape(T * V, D)

    pe2d = pe.reshape(pe.shape[1], D)[:S]

    out = _make_sc_call(B, S, V, D)(tok, trk, ctab, pe2d)
    return out.reshape(B, S, D)
